# baseline (speedup 1.0000x reference)
import functools

import jax
import jax.numpy as jnp
from jax import lax
from jax.experimental import pallas as pl
from jax.experimental.pallas import tpu as pltpu
from jax.experimental.pallas import tpu_sc as plsc

NUM_WORKERS = 32
CHUNK = 800


@functools.partial(jax.jit, static_argnames=())
def _embedding_gather(idx, table):
    n = idx.shape[0]
    _, d = table.shape
    per_w = n // NUM_WORKERS
    k = per_w // CHUNK

    mesh = plsc.VectorSubcoreMesh(core_axis_name="c", subcore_axis_name="s")

    @functools.partial(
        pl.kernel,
        mesh=mesh,
        out_type=jax.ShapeDtypeStruct((n, d), jnp.float32),
        scratch_types=[
            pltpu.VMEM((per_w,), jnp.int32),
            pltpu.VMEM((CHUNK, d), jnp.float32),
            pltpu.VMEM((CHUNK, d), jnp.float32),
            pltpu.SemaphoreType.DMA,
            pltpu.SemaphoreType.DMA,
        ],
        compiler_params=pltpu.CompilerParams(use_tc_tiling_on_sc=False),
    )
    def body(idx_hbm, table_hbm, out_hbm, idx_v, buf0, buf1, gsem, osem):
        wid = lax.axis_index("s") * 2 + lax.axis_index("c")
        base = wid * per_w
        pltpu.sync_copy(idx_hbm.at[pl.ds(base, per_w)], idx_v)
        bufs = (buf0, buf1)
        cp0 = pltpu.async_copy(
            table_hbm.at[idx_v.at[pl.ds(0, CHUNK)]], bufs[0], gsem
        )
        pending = [cp0]
        out_pending = [None, None]
        for j in range(k):
            buf = bufs[j % 2]
            pending[0].wait()
            if j + 1 < k:
                nbuf = bufs[(j + 1) % 2]
                if out_pending[(j + 1) % 2] is not None:
                    out_pending[(j + 1) % 2].wait()
                    out_pending[(j + 1) % 2] = None
                pending[0] = pltpu.async_copy(
                    table_hbm.at[idx_v.at[pl.ds((j + 1) * CHUNK, CHUNK)]],
                    nbuf,
                    gsem,
                )
            out_pending[j % 2] = pltpu.async_copy(
                buf, out_hbm.at[pl.ds(base + j * CHUNK, CHUNK)], osem
            )
        for cp in out_pending:
            if cp is not None:
                cp.wait()

    return body(idx, table)


def kernel(x, table):
    b, l = x.shape
    _, d = table.shape
    idx = x.reshape(b * l).astype(jnp.int32)
    out = _embedding_gather(idx, table)
    return out.reshape(b, l, d)
